# baseline (speedup 1.0000x reference)
import functools

import jax
import jax.numpy as jnp
from jax import lax
from jax.experimental import pallas as pl
from jax.experimental.pallas import tpu as pltpu
from jax.experimental.pallas import tpu_sc as plsc

E = 8
H = 2048
D = 1408
T = 2048
K = 2
A = T * K
BM = 512
SMAX = 8192
NB = SMAX // BM

SC_NC = 2
SC_NS = 16
SC_NW = SC_NC * SC_NS
CH = 16
NCHUNK = A // (SC_NW * CH)


def _cumsum(x, axis):
    n = x.shape[axis]
    c = x
    s = 1
    while s < n:
        if axis == 0:
            z = jnp.zeros((s,) + x.shape[1:], x.dtype)
            c = c + jnp.concatenate([z, c[: n - s]], axis=0)
        else:
            z = jnp.zeros(x.shape[:1] + (s,), x.dtype)
            c = c + jnp.concatenate([z, c[:, : n - s]], axis=1)
        s *= 2
    return c


def _router_body(h_ref, g_ref, pos_ref, src_ref, w_ref, be_ref):
    h = h_ref[...]
    g = g_ref[...]

    logits = lax.dot_general(h, g, (((1,), (1,)), ((), ())),
                             preferred_element_type=jnp.float32)
    m1 = jnp.max(logits, axis=1, keepdims=True)
    is1 = logits == m1
    is1 = jnp.logical_and(is1, _cumsum(is1.astype(jnp.int32), axis=1) == 1)
    masked = jnp.where(is1, -jnp.inf, logits)
    m2 = jnp.max(masked, axis=1, keepdims=True)
    w1 = 1.0 / (1.0 + jnp.exp(m2 - m1))
    w_ref[...] = jnp.concatenate([w1, 1.0 - w1], axis=0)

    lt = lax.dot_general(g, h, (((1,), (1,)), ((), ())),
                         preferred_element_type=jnp.float32)
    m1t = jnp.max(lt, axis=0, keepdims=True)
    is1t = lt == m1t
    is1t = jnp.logical_and(is1t, _cumsum(is1t.astype(jnp.int32), axis=0) == 1)
    maskedt = jnp.where(is1t, -jnp.inf, lt)
    m2t = jnp.max(maskedt, axis=0, keepdims=True)
    is2t = maskedt == m2t
    is2t = jnp.logical_and(is2t, _cumsum(is2t.astype(jnp.int32), axis=0) == 1)
    iota_e = lax.broadcasted_iota(jnp.int32, (E, T), 0)
    e1 = jnp.sum(jnp.where(is1t, iota_e, 0), axis=0, keepdims=True)
    e2 = jnp.sum(jnp.where(is2t, iota_e, 0), axis=0, keepdims=True)

    a = jnp.concatenate([e1, e2], axis=1)
    onehot = (a == lax.broadcasted_iota(jnp.int32, (E, A), 0)).astype(jnp.int32)
    csum = _cumsum(onehot, axis=1)
    rank = jnp.sum(csum * onehot, axis=0, keepdims=True) - 1
    counts = csum[:, A - 1:A]
    padded = ((counts + BM - 1) // BM) * BM
    ends = _cumsum(padded, axis=0)
    offs = ends - padded
    off_i = jnp.sum(onehot * offs, axis=0, keepdims=True)
    pos_ref[...] = off_i + rank
    src_ref[...] = lax.broadcasted_iota(jnp.int32, (1, A), 1) % T
    bstart = lax.broadcasted_iota(jnp.int32, (1, NB), 1) * BM
    be = jnp.sum((bstart >= ends).astype(jnp.int32), axis=0, keepdims=True)
    be_ref[...] = jnp.minimum(be, E - 1)


def _router_call(hidden, gate, interpret=False):
    return pl.pallas_call(
        _router_body,
        out_shape=(
            jax.ShapeDtypeStruct((1, A), jnp.int32),
            jax.ShapeDtypeStruct((1, A), jnp.int32),
            jax.ShapeDtypeStruct((A, 1), jnp.float32),
            jax.ShapeDtypeStruct((1, NB), jnp.int32),
        ),
        interpret=interpret,
    )(hidden, gate)


def _sc_mesh():
    return plsc.VectorSubcoreMesh(core_axis_name="c", subcore_axis_name="s")


def _dispatch_call(hidden, src, pos):
    @functools.partial(
        pl.kernel,
        mesh=_sc_mesh(),
        out_type=jax.ShapeDtypeStruct((SMAX, H), jnp.float32),
        scratch_types=[
            pltpu.VMEM((CH,), jnp.int32),
            pltpu.VMEM((CH,), jnp.int32),
            pltpu.VMEM((CH,), jnp.int32),
            pltpu.VMEM((CH,), jnp.int32),
            pltpu.VMEM((CH, H), jnp.float32),
            pltpu.VMEM((CH, H), jnp.float32),
            pltpu.SemaphoreType.DMA,
            pltpu.SemaphoreType.DMA,
            pltpu.SemaphoreType.DMA,
            pltpu.SemaphoreType.DMA,
        ],
    )
    def k(hid_hbm, src_hbm, pos_hbm, xs_hbm,
          srcv0, srcv1, posv0, posv1, rows0, rows1,
          gsem0, gsem1, ssem0, ssem1):
        wid = lax.axis_index("s") * SC_NC + lax.axis_index("c")
        srcv, posv = [srcv0, srcv1], [posv0, posv1]
        rows, gs, ss = [rows0, rows1], [gsem0, gsem1], [ssem0, ssem1]

        def load_idx(c, b):
            base = wid * (CH * NCHUNK) + c * CH
            pltpu.sync_copy(src_hbm.at[pl.ds(base, CH)], srcv[b])
            pltpu.sync_copy(pos_hbm.at[pl.ds(base, CH)], posv[b])

        gh = [None] * NCHUNK
        sh = [None] * NCHUNK
        load_idx(0, 0)
        gh[0] = pltpu.async_copy(hid_hbm.at[srcv0], rows0, gsem0)
        for c in range(NCHUNK):
            b = c % 2
            if c + 1 < NCHUNK:
                bn = (c + 1) % 2
                if c >= 1:
                    sh[c - 1].wait()
                load_idx(c + 1, bn)
                gh[c + 1] = pltpu.async_copy(
                    hid_hbm.at[srcv[bn]], rows[bn], gs[bn])
            gh[c].wait()
            sh[c] = pltpu.async_copy(rows[b], xs_hbm.at[posv[b]], ss[b])
        if NCHUNK >= 2:
            sh[NCHUNK - 2].wait()
        sh[NCHUNK - 1].wait()

    return k(hidden, src, pos)


def _gather_call(y, pos):
    @functools.partial(
        pl.kernel,
        mesh=_sc_mesh(),
        out_type=jax.ShapeDtypeStruct((A, H), jnp.float32),
        scratch_types=[
            pltpu.VMEM((CH,), jnp.int32),
            pltpu.VMEM((CH,), jnp.int32),
            pltpu.VMEM((CH, H), jnp.float32),
            pltpu.VMEM((CH, H), jnp.float32),
            pltpu.SemaphoreType.DMA,
            pltpu.SemaphoreType.DMA,
            pltpu.SemaphoreType.DMA,
            pltpu.SemaphoreType.DMA,
        ],
    )
    def k(y_hbm, pos_hbm, out_hbm,
          posv0, posv1, rows0, rows1, gsem0, gsem1, wsem0, wsem1):
        wid = lax.axis_index("s") * SC_NC + lax.axis_index("c")
        posv = [posv0, posv1]
        rows, gs, ws = [rows0, rows1], [gsem0, gsem1], [wsem0, wsem1]

        def base(c):
            return wid * (CH * NCHUNK) + c * CH

        gh = [None] * NCHUNK
        wh = [None] * NCHUNK
        pltpu.sync_copy(pos_hbm.at[pl.ds(base(0), CH)], posv0)
        gh[0] = pltpu.async_copy(y_hbm.at[posv0], rows0, gsem0)
        for c in range(NCHUNK):
            b = c % 2
            if c + 1 < NCHUNK:
                bn = (c + 1) % 2
                if c >= 1:
                    wh[c - 1].wait()
                pltpu.sync_copy(pos_hbm.at[pl.ds(base(c + 1), CH)], posv[bn])
                gh[c + 1] = pltpu.async_copy(
                    y_hbm.at[posv[bn]], rows[bn], gs[bn])
            gh[c].wait()
            wh[c] = pltpu.async_copy(
                rows[b], out_hbm.at[pl.ds(base(c), CH)], ws[b])
        if NCHUNK >= 2:
            wh[NCHUNK - 2].wait()
        wh[NCHUNK - 1].wait()

    return k(y, pos)


def _mm1_body(be_ref, xs_ref, w_ref, gu_ref, w_bf):
    b = pl.program_id(1)
    prev = be_ref[jnp.maximum(b - 1, 0)]
    cur = be_ref[b]

    @pl.when(jnp.logical_or(b == 0, prev != cur))
    def _():
        w_bf[...] = w_ref[0].astype(jnp.bfloat16)

    w = w_bf[...]
    for i in range(0, BM, 128):
        x = xs_ref[i:i + 128, :].astype(jnp.bfloat16)
        gu = jnp.dot(x, w, preferred_element_type=jnp.float32)
        gu_ref[i:i + 128, :] = gu.astype(jnp.bfloat16)


def _mm1_call(xs, gup, be, interpret=False):
    grid_spec = pltpu.PrefetchScalarGridSpec(
        num_scalar_prefetch=1,
        grid=(2, NB),
        in_specs=[
            pl.BlockSpec((BM, H), lambda n, b, be: (b, 0)),
            pl.BlockSpec((1, H, D), lambda n, b, be: (be[b], 0, n)),
        ],
        out_specs=pl.BlockSpec((BM, D), lambda n, b, be: (b, n)),
        scratch_shapes=[pltpu.VMEM((H, D), jnp.bfloat16)],
    )
    return pl.pallas_call(
        _mm1_body,
        grid_spec=grid_spec,
        out_shape=jax.ShapeDtypeStruct((SMAX, 2 * D), jnp.bfloat16),
        interpret=interpret,
    )(be, xs, gup)


def _mm2_body(be_ref, g_ref, u_ref, w_ref, y_ref, w_bf):
    b = pl.program_id(0)
    prev = be_ref[jnp.maximum(b - 1, 0)]
    cur = be_ref[b]

    @pl.when(jnp.logical_or(b == 0, prev != cur))
    def _():
        w_bf[...] = w_ref[0].astype(jnp.bfloat16)

    w = w_bf[...]
    for i in range(0, BM, 128):
        g = g_ref[i:i + 128, :].astype(jnp.float32)
        u = u_ref[i:i + 128, :].astype(jnp.float32)
        act = (g * jax.nn.sigmoid(g) * u).astype(jnp.bfloat16)
        y_ref[i:i + 128, :] = jnp.dot(
            act, w, preferred_element_type=jnp.float32)


def _mm2_call(gu, down, be, interpret=False):
    grid_spec = pltpu.PrefetchScalarGridSpec(
        num_scalar_prefetch=1,
        grid=(NB,),
        in_specs=[
            pl.BlockSpec((BM, D), lambda b, be: (b, 0)),
            pl.BlockSpec((BM, D), lambda b, be: (b, 1)),
            pl.BlockSpec((1, D, H), lambda b, be: (be[b], 0, 0)),
        ],
        out_specs=pl.BlockSpec((BM, H), lambda b, be: (b, 0)),
        scratch_shapes=[pltpu.VMEM((D, H), jnp.bfloat16)],
    )
    return pl.pallas_call(
        _mm2_body,
        grid_spec=grid_spec,
        out_shape=jax.ShapeDtypeStruct((SMAX, H), jnp.float32),
        interpret=interpret,
    )(be, gu, gu, down)


BT = 256


def _combine_body(y1_ref, y2_ref, w1_ref, w2_ref, out_ref):
    out_ref[...] = w1_ref[...] * y1_ref[...] + w2_ref[...] * y2_ref[...]


def _combine_call(yg, wflat, interpret=False):
    nt = T // BT
    return pl.pallas_call(
        _combine_body,
        grid=(nt,),
        in_specs=[
            pl.BlockSpec((BT, H), lambda t: (t, 0)),
            pl.BlockSpec((BT, H), lambda t: (t + nt, 0)),
            pl.BlockSpec((BT, 1), lambda t: (t, 0)),
            pl.BlockSpec((BT, 1), lambda t: (t + nt, 0)),
        ],
        out_specs=pl.BlockSpec((BT, H), lambda t: (t, 0)),
        out_shape=jax.ShapeDtypeStruct((T, H), jnp.float32),
        interpret=interpret,
    )(yg, yg, wflat, wflat)


def kernel(hidden_states, gate_weight, gate_up_proj, down_proj):
    pos2, src2, wflat, be2 = _router_call(hidden_states, gate_weight)
    pos = pos2.reshape(A)
    src = src2.reshape(A)
    be = be2.reshape(NB)

    xs = _dispatch_call(hidden_states, src, pos)
    gu = _mm1_call(xs, gate_up_proj, be)
    y = _mm2_call(gu, down_proj, be)
    yg = _gather_call(y, pos)
    return _combine_call(yg, wflat)


# speedup vs baseline: 1.1002x; 1.1002x over previous
import functools

import jax
import jax.numpy as jnp
from jax import lax
from jax.experimental import pallas as pl
from jax.experimental.pallas import tpu as pltpu
from jax.experimental.pallas import tpu_sc as plsc

E = 8
H = 2048
D = 1408
T = 2048
K = 2
A = T * K
BM = 128
SMAX = 5120
NB = SMAX // BM

SC_NC = 2
SC_NS = 16
SC_NW = SC_NC * SC_NS
CH = 16
NCHUNK = A // (SC_NW * CH)


def _cumsum(x, axis):
    n = x.shape[axis]
    c = x
    s = 1
    while s < n:
        if axis == 0:
            z = jnp.zeros((s,) + x.shape[1:], x.dtype)
            c = c + jnp.concatenate([z, c[: n - s]], axis=0)
        else:
            z = jnp.zeros(x.shape[:1] + (s,), x.dtype)
            c = c + jnp.concatenate([z, c[:, : n - s]], axis=1)
        s *= 2
    return c


def _router_body(h_ref, g_ref, pos_ref, src_ref, w_ref, be_ref):
    h = h_ref[...]
    g = g_ref[...]

    logits = lax.dot_general(h, g, (((1,), (1,)), ((), ())),
                             preferred_element_type=jnp.float32)
    m1 = jnp.max(logits, axis=1, keepdims=True)
    is1 = logits == m1
    is1 = jnp.logical_and(is1, _cumsum(is1.astype(jnp.int32), axis=1) == 1)
    masked = jnp.where(is1, -jnp.inf, logits)
    m2 = jnp.max(masked, axis=1, keepdims=True)
    w1 = 1.0 / (1.0 + jnp.exp(m2 - m1))
    w_ref[...] = jnp.concatenate([w1, 1.0 - w1], axis=0)

    lt = lax.dot_general(g, h, (((1,), (1,)), ((), ())),
                         preferred_element_type=jnp.float32)
    m1t = jnp.max(lt, axis=0, keepdims=True)
    is1t = lt == m1t
    is1t = jnp.logical_and(is1t, _cumsum(is1t.astype(jnp.int32), axis=0) == 1)
    maskedt = jnp.where(is1t, -jnp.inf, lt)
    m2t = jnp.max(maskedt, axis=0, keepdims=True)
    is2t = maskedt == m2t
    is2t = jnp.logical_and(is2t, _cumsum(is2t.astype(jnp.int32), axis=0) == 1)
    iota_e = lax.broadcasted_iota(jnp.int32, (E, T), 0)
    e1 = jnp.sum(jnp.where(is1t, iota_e, 0), axis=0, keepdims=True)
    e2 = jnp.sum(jnp.where(is2t, iota_e, 0), axis=0, keepdims=True)

    a = jnp.concatenate([e1, e2], axis=1)
    onehot = (a == lax.broadcasted_iota(jnp.int32, (E, A), 0)).astype(jnp.int32)
    csum = _cumsum(onehot, axis=1)
    rank = jnp.sum(csum * onehot, axis=0, keepdims=True) - 1
    counts = csum[:, A - 1:A]
    padded = ((counts + BM - 1) // BM) * BM
    ends = _cumsum(padded, axis=0)
    offs = ends - padded
    off_i = jnp.sum(onehot * offs, axis=0, keepdims=True)
    pos_ref[...] = off_i + rank
    src_ref[...] = lax.broadcasted_iota(jnp.int32, (1, A), 1) % T
    bstart = lax.broadcasted_iota(jnp.int32, (1, NB), 1) * BM
    be = jnp.sum((bstart >= ends).astype(jnp.int32), axis=0, keepdims=True)
    be_ref[...] = jnp.minimum(be, E - 1)


def _router_call(hidden, gate, interpret=False):
    return pl.pallas_call(
        _router_body,
        out_shape=(
            jax.ShapeDtypeStruct((1, A), jnp.int32),
            jax.ShapeDtypeStruct((1, A), jnp.int32),
            jax.ShapeDtypeStruct((A, 1), jnp.float32),
            jax.ShapeDtypeStruct((1, NB), jnp.int32),
        ),
        interpret=interpret,
    )(hidden, gate)


def _sc_mesh():
    return plsc.VectorSubcoreMesh(core_axis_name="c", subcore_axis_name="s")


def _dispatch_call(hidden, src, pos):
    @functools.partial(
        pl.kernel,
        mesh=_sc_mesh(),
        out_type=jax.ShapeDtypeStruct((SMAX, H), jnp.float32),
        scratch_types=[
            pltpu.VMEM((CH,), jnp.int32),
            pltpu.VMEM((CH,), jnp.int32),
            pltpu.VMEM((CH,), jnp.int32),
            pltpu.VMEM((CH,), jnp.int32),
            pltpu.VMEM((CH, H), jnp.float32),
            pltpu.VMEM((CH, H), jnp.float32),
            pltpu.SemaphoreType.DMA,
            pltpu.SemaphoreType.DMA,
            pltpu.SemaphoreType.DMA,
            pltpu.SemaphoreType.DMA,
        ],
    )
    def k(hid_hbm, src_hbm, pos_hbm, xs_hbm,
          srcv0, srcv1, posv0, posv1, rows0, rows1,
          gsem0, gsem1, ssem0, ssem1):
        wid = lax.axis_index("s") * SC_NC + lax.axis_index("c")
        srcv, posv = [srcv0, srcv1], [posv0, posv1]
        rows, gs, ss = [rows0, rows1], [gsem0, gsem1], [ssem0, ssem1]

        def load_idx(c, b):
            base = wid * (CH * NCHUNK) + c * CH
            pltpu.sync_copy(src_hbm.at[pl.ds(base, CH)], srcv[b])
            pltpu.sync_copy(pos_hbm.at[pl.ds(base, CH)], posv[b])

        gh = [None] * NCHUNK
        sh = [None] * NCHUNK
        load_idx(0, 0)
        gh[0] = pltpu.async_copy(hid_hbm.at[srcv0], rows0, gsem0)
        for c in range(NCHUNK):
            b = c % 2
            if c + 1 < NCHUNK:
                bn = (c + 1) % 2
                if c >= 1:
                    sh[c - 1].wait()
                load_idx(c + 1, bn)
                gh[c + 1] = pltpu.async_copy(
                    hid_hbm.at[srcv[bn]], rows[bn], gs[bn])
            gh[c].wait()
            sh[c] = pltpu.async_copy(rows[b], xs_hbm.at[posv[b]], ss[b])
        if NCHUNK >= 2:
            sh[NCHUNK - 2].wait()
        sh[NCHUNK - 1].wait()

    return k(hidden, src, pos)


def _gather_call(y, pos):
    @functools.partial(
        pl.kernel,
        mesh=_sc_mesh(),
        out_type=jax.ShapeDtypeStruct((A, H), jnp.float32),
        scratch_types=[
            pltpu.VMEM((CH,), jnp.int32),
            pltpu.VMEM((CH,), jnp.int32),
            pltpu.VMEM((CH, H), jnp.float32),
            pltpu.VMEM((CH, H), jnp.float32),
            pltpu.SemaphoreType.DMA,
            pltpu.SemaphoreType.DMA,
            pltpu.SemaphoreType.DMA,
            pltpu.SemaphoreType.DMA,
        ],
    )
    def k(y_hbm, pos_hbm, out_hbm,
          posv0, posv1, rows0, rows1, gsem0, gsem1, wsem0, wsem1):
        wid = lax.axis_index("s") * SC_NC + lax.axis_index("c")
        posv = [posv0, posv1]
        rows, gs, ws = [rows0, rows1], [gsem0, gsem1], [wsem0, wsem1]

        def base(c):
            return wid * (CH * NCHUNK) + c * CH

        gh = [None] * NCHUNK
        wh = [None] * NCHUNK
        pltpu.sync_copy(pos_hbm.at[pl.ds(base(0), CH)], posv0)
        gh[0] = pltpu.async_copy(y_hbm.at[posv0], rows0, gsem0)
        for c in range(NCHUNK):
            b = c % 2
            if c + 1 < NCHUNK:
                bn = (c + 1) % 2
                if c >= 1:
                    wh[c - 1].wait()
                pltpu.sync_copy(pos_hbm.at[pl.ds(base(c + 1), CH)], posv[bn])
                gh[c + 1] = pltpu.async_copy(
                    y_hbm.at[posv[bn]], rows[bn], gs[bn])
            gh[c].wait()
            wh[c] = pltpu.async_copy(
                rows[b], out_hbm.at[pl.ds(base(c), CH)], ws[b])
        if NCHUNK >= 2:
            wh[NCHUNK - 2].wait()
        wh[NCHUNK - 1].wait()

    return k(y, pos)


def _mm1_body(be_ref, xs_ref, w_ref, gu_ref):
    gu_ref[...] = jnp.dot(xs_ref[...], w_ref[0],
                          preferred_element_type=jnp.float32)


def _mm1_call(xs, gup, be, interpret=False):
    grid_spec = pltpu.PrefetchScalarGridSpec(
        num_scalar_prefetch=1,
        grid=(NB,),
        in_specs=[
            pl.BlockSpec((BM, H), lambda b, be: (b, 0)),
            pl.BlockSpec((1, H, 2 * D), lambda b, be: (be[b], 0, 0)),
        ],
        out_specs=pl.BlockSpec((BM, 2 * D), lambda b, be: (b, 0)),
    )
    return pl.pallas_call(
        _mm1_body,
        grid_spec=grid_spec,
        out_shape=jax.ShapeDtypeStruct((SMAX, 2 * D), jnp.float32),
        interpret=interpret,
    )(be, xs, gup)


def _mm2_body(be_ref, g_ref, u_ref, w_ref, y_ref):
    g = g_ref[...]
    u = u_ref[...]
    act = g * jax.nn.sigmoid(g) * u
    y_ref[...] = jnp.dot(act, w_ref[0], preferred_element_type=jnp.float32)


def _mm2_call(gu, down, be, interpret=False):
    grid_spec = pltpu.PrefetchScalarGridSpec(
        num_scalar_prefetch=1,
        grid=(NB,),
        in_specs=[
            pl.BlockSpec((BM, D), lambda b, be: (b, 0)),
            pl.BlockSpec((BM, D), lambda b, be: (b, 1)),
            pl.BlockSpec((1, D, H), lambda b, be: (be[b], 0, 0)),
        ],
        out_specs=pl.BlockSpec((BM, H), lambda b, be: (b, 0)),
    )
    return pl.pallas_call(
        _mm2_body,
        grid_spec=grid_spec,
        out_shape=jax.ShapeDtypeStruct((SMAX, H), jnp.float32),
        interpret=interpret,
    )(be, gu, gu, down)


BT = 256


def _combine_body(y1_ref, y2_ref, w1_ref, w2_ref, out_ref):
    out_ref[...] = w1_ref[...] * y1_ref[...] + w2_ref[...] * y2_ref[...]


def _combine_call(yg, wflat, interpret=False):
    nt = T // BT
    return pl.pallas_call(
        _combine_body,
        grid=(nt,),
        in_specs=[
            pl.BlockSpec((BT, H), lambda t: (t, 0)),
            pl.BlockSpec((BT, H), lambda t: (t + nt, 0)),
            pl.BlockSpec((BT, 1), lambda t: (t, 0)),
            pl.BlockSpec((BT, 1), lambda t: (t + nt, 0)),
        ],
        out_specs=pl.BlockSpec((BT, H), lambda t: (t, 0)),
        out_shape=jax.ShapeDtypeStruct((T, H), jnp.float32),
        interpret=interpret,
    )(yg, yg, wflat, wflat)


def kernel(hidden_states, gate_weight, gate_up_proj, down_proj):
    pos2, src2, wflat, be2 = _router_call(hidden_states, gate_weight)
    pos = pos2.reshape(A)
    src = src2.reshape(A)
    be = be2.reshape(NB)

    xs = _dispatch_call(hidden_states, src, pos)
    gu = _mm1_call(xs, gate_up_proj, be)
    y = _mm2_call(gu, down_proj, be)
    yg = _gather_call(y, pos)
    return _combine_call(yg, wflat)
